# baseline (speedup 1.0000x reference)
import functools

import jax
import jax.numpy as jnp
from jax import lax
from jax.experimental import pallas as pl

_F32 = jnp.float32
_EPS = 1e-5


def _pick_bm(m, cap=512):
    for c in (cap, 384, 256, 192, 128, 96, 64, 32, 16, 8):
        if c <= cap and m % c == 0:
            return c
    return m


def _mm_body(*refs, leaky, has_add):
    if has_add:
        x_ref, w_ref, m_ref, sv_ref, g_ref, b_ref, a_ref, o_ref = refs
    else:
        x_ref, w_ref, m_ref, sv_ref, g_ref, b_ref, o_ref = refs
    y = jnp.dot(x_ref[...], w_ref[...], preferred_element_type=_F32)
    y = (y - m_ref[0, :]) / sv_ref[0, :] * g_ref[0, :] + b_ref[0, :]
    if leaky:
        y = jnp.where(y > 0, y, 0.1 * y)
    if has_add:
        y = y + a_ref[...]
    o_ref[...] = y


def _matmul_fused(x2d, w, bn_m, bn_sv, bn_g, bn_b, *, leaky=True, add=None):
    m, k = x2d.shape
    n = w.shape[1]
    bm = _pick_bm(m)
    bn = min(n, 256)
    grid = (m // bm, n // bn)
    vec_spec = pl.BlockSpec((1, bn), lambda i, j: (0, j))
    in_specs = [
        pl.BlockSpec((bm, k), lambda i, j: (i, 0)),
        pl.BlockSpec((k, bn), lambda i, j: (0, j)),
        vec_spec, vec_spec, vec_spec, vec_spec,
    ]
    args = [x2d, w, bn_m.reshape(1, n), bn_sv.reshape(1, n),
            bn_g.reshape(1, n), bn_b.reshape(1, n)]
    if add is not None:
        in_specs.append(pl.BlockSpec((bm, bn), lambda i, j: (i, j)))
        args.append(add)
    return pl.pallas_call(
        functools.partial(_mm_body, leaky=leaky, has_add=add is not None),
        grid=grid,
        in_specs=in_specs,
        out_specs=pl.BlockSpec((bm, bn), lambda i, j: (i, j)),
        out_shape=jax.ShapeDtypeStruct((m, n), _F32),
    )(*args)


def _im2col(x, stride):
    nb, h, wd, c = x.shape
    xp = jnp.pad(x, ((0, 0), (1, 1), (1, 1), (0, 0)))
    ho, wo = h // stride, wd // stride
    cols = [
        xp[:, ky:ky + stride * (ho - 1) + 1:stride,
           kx:kx + stride * (wo - 1) + 1:stride, :]
        for ky in range(3) for kx in range(3)
    ]
    return jnp.concatenate(cols, axis=-1), ho, wo


def _cbl_x(x, p, stride=1, add=None):
    w = p["w"]
    o, i, k, _ = w.shape
    sv = jnp.sqrt(p["v"] + _EPS)
    bn_args = (p["m"], sv, p["g"], p["b"])
    nb, h, wd, _ = x.shape
    if k == 1:
        w2 = w.reshape(o, i).T
        a2 = add.reshape(-1, o) if add is not None else None
        y = _matmul_fused(x.reshape(-1, i), w2, *bn_args, add=a2)
        return y.reshape(nb, h, wd, o)
    cols, ho, wo = _im2col(x, stride)
    w2 = jnp.transpose(w, (2, 3, 1, 0)).reshape(9 * i, o)
    a2 = add.reshape(-1, o) if add is not None else None
    y = _matmul_fused(cols.reshape(-1, 9 * i), w2, *bn_args, add=a2)
    return y.reshape(nb, ho, wo, o)


def _seq_x(x, ps):
    for p in ps:
        x = _cbl_x(x, p)
    return x


def _up2(x):
    return jnp.repeat(jnp.repeat(x, 2, axis=1), 2, axis=2)


def _detect_x(x, dp):
    y = _cbl_x(x, dp["c"])
    w = dp["d"]["w"][:, :, 0, 0].T
    i, o = w.shape
    pad_n = 128 - o
    wp = jnp.pad(w, ((0, 0), (0, pad_n)))
    zeros = jnp.zeros((128,), _F32)
    ones = jnp.ones((128,), _F32)
    bp = jnp.pad(dp["d"]["b"], (0, pad_n))
    nb, h, wd, _ = y.shape
    z = _matmul_fused(y.reshape(-1, i), wp, zeros, ones, ones, bp, leaky=False)
    return z.reshape(nb, h, wd, 128)[..., :o]


def kernel(x, params):
    xh = jnp.transpose(x, (0, 2, 3, 1))
    t = _cbl_x(xh, params["stem"])
    feats = []
    for st in params["stages"]:
        t = _cbl_x(t, st["down"], stride=2)
        for blk in st["blocks"]:
            t = _cbl_x(_cbl_x(t, blk["c1"]), blk["c2"], add=t)
        feats.append(t)
    f13, f26, f52 = feats[-1], feats[-2], feats[-3]

    x1 = _seq_x(f13, params["s1"])
    p13 = _detect_x(x1, params["s1d"])
    x1u = _up2(_cbl_x(x1, params["up1"]))
    x2 = _seq_x(jnp.concatenate([x1u, f26], axis=-1), params["s2"])
    p26 = _detect_x(x2, params["s2d"])
    x2u = _up2(_cbl_x(x2, params["up2"]))
    x3 = _seq_x(jnp.concatenate([x2u, f52], axis=-1), params["s3"])
    p52 = _detect_x(x3, params["s3d"])
    return p13, p26, p52


# speedup vs baseline: 1.2271x; 1.2271x over previous
import functools

import jax
import jax.numpy as jnp
from jax import lax
from jax.experimental import pallas as pl

_F32 = jnp.float32
_EPS = 1e-5


def _pick_bm(m, cap=512):
    for c in (cap, 384, 256, 192, 128, 96, 64, 32, 16, 8):
        if c <= cap and m % c == 0:
            return c
    return m


def _mm_body(*refs, leaky, has_add):
    if has_add:
        x_ref, w_ref, m_ref, sv_ref, g_ref, b_ref, a_ref, o_ref = refs
    else:
        x_ref, w_ref, m_ref, sv_ref, g_ref, b_ref, o_ref = refs
    y = jnp.dot(x_ref[...], w_ref[...], preferred_element_type=_F32)
    y = (y - m_ref[0, :]) / sv_ref[0, :] * g_ref[0, :] + b_ref[0, :]
    if leaky:
        y = jnp.where(y > 0, y, 0.1 * y)
    if has_add:
        y = y + a_ref[...]
    o_ref[...] = y


def _matmul_fused(x2d, w, bn_m, bn_sv, bn_g, bn_b, *, leaky=True, add=None):
    m, k = x2d.shape
    n = w.shape[1]
    bm = _pick_bm(m)
    bn = min(n, 256)
    grid = (m // bm, n // bn)
    vec_spec = pl.BlockSpec((1, bn), lambda i, j: (0, j))
    in_specs = [
        pl.BlockSpec((bm, k), lambda i, j: (i, 0)),
        pl.BlockSpec((k, bn), lambda i, j: (0, j)),
        vec_spec, vec_spec, vec_spec, vec_spec,
    ]
    args = [x2d, w, bn_m.reshape(1, n), bn_sv.reshape(1, n),
            bn_g.reshape(1, n), bn_b.reshape(1, n)]
    if add is not None:
        in_specs.append(pl.BlockSpec((bm, bn), lambda i, j: (i, j)))
        args.append(add)
    return pl.pallas_call(
        functools.partial(_mm_body, leaky=leaky, has_add=add is not None),
        grid=grid,
        in_specs=in_specs,
        out_specs=pl.BlockSpec((bm, bn), lambda i, j: (i, j)),
        out_shape=jax.ShapeDtypeStruct((m, n), _F32),
    )(*args)


def _c3_body(*refs, bh, wdim, leaky, has_add):
    if has_add:
        x_ref, w_ref, m_ref, sv_ref, g_ref, b_ref, a_ref, o_ref = refs
    else:
        x_ref, w_ref, m_ref, sv_ref, g_ref, b_ref, o_ref = refs
    row0 = pl.program_id(1) * bh
    bn = o_ref.shape[-1]
    acc = jnp.zeros((bh, wdim, bn), _F32)
    for ky in range(3):
        for kx in range(3):
            xs = x_ref[0, pl.ds(row0 + ky, bh), pl.ds(kx, wdim), :]
            acc = acc + lax.dot_general(xs, w_ref[ky * 3 + kx],
                                        (((2,), (0,)), ((), ())),
                                        preferred_element_type=_F32)
    y = (acc - m_ref[0, :]) / sv_ref[0, :] * g_ref[0, :] + b_ref[0, :]
    if leaky:
        y = jnp.where(y > 0, y, 0.1 * y)
    if has_add:
        y = y + a_ref[0]
    o_ref[0] = y


def _conv3_fused(x, w9, bn_m, bn_sv, bn_g, bn_b, *, leaky=True, add=None):
    nb, h, wd, cin = x.shape
    cout = w9.shape[-1]
    xp = jnp.pad(x, ((0, 0), (1, 1), (1, 1), (0, 0)))
    bh = h
    for c in (16, 8, 12, 24):
        if h % c == 0 and c * wd * max(cin, cout) <= 1 << 19:
            bh = c
            break
    bn = min(cout, 256)
    grid = (nb, h // bh, cout // bn)
    vec_spec = pl.BlockSpec((1, bn), lambda n_, r, c: (0, c))
    n_ = cout
    in_specs = [
        pl.BlockSpec((1, h + 2, wd + 2, cin), lambda n_, r, c: (n_, 0, 0, 0)),
        pl.BlockSpec((9, cin, bn), lambda n_, r, c: (0, 0, c)),
        vec_spec, vec_spec, vec_spec, vec_spec,
    ]
    args = [xp, w9, bn_m.reshape(1, cout), bn_sv.reshape(1, cout),
            bn_g.reshape(1, cout), bn_b.reshape(1, cout)]
    if add is not None:
        in_specs.append(pl.BlockSpec((1, bh, wd, bn), lambda n_, r, c: (n_, r, 0, c)))
        args.append(add)
    return pl.pallas_call(
        functools.partial(_c3_body, bh=bh, wdim=wd, leaky=leaky,
                          has_add=add is not None),
        grid=grid,
        in_specs=in_specs,
        out_specs=pl.BlockSpec((1, bh, wd, bn), lambda n_, r, c: (n_, r, 0, c)),
        out_shape=jax.ShapeDtypeStruct((nb, h, wd, cout), _F32),
    )(*args)


def _im2col(x, stride):
    nb, h, wd, c = x.shape
    xp = jnp.pad(x, ((0, 0), (1, 1), (1, 1), (0, 0)))
    ho, wo = h // stride, wd // stride
    cols = [
        xp[:, ky:ky + stride * (ho - 1) + 1:stride,
           kx:kx + stride * (wo - 1) + 1:stride, :]
        for ky in range(3) for kx in range(3)
    ]
    return jnp.concatenate(cols, axis=-1), ho, wo


def _cbl_x(x, p, stride=1, add=None):
    w = p["w"]
    o, i, k, _ = w.shape
    sv = jnp.sqrt(p["v"] + _EPS)
    bn_args = (p["m"], sv, p["g"], p["b"])
    nb, h, wd, _ = x.shape
    if k == 1:
        w2 = w.reshape(o, i).T
        a2 = add.reshape(-1, o) if add is not None else None
        y = _matmul_fused(x.reshape(-1, i), w2, *bn_args, add=a2)
        return y.reshape(nb, h, wd, o)
    if stride == 1 and i >= 32:
        w9 = jnp.transpose(w, (2, 3, 1, 0)).reshape(9, i, o)
        return _conv3_fused(x, w9, *bn_args, add=add)
    cols, ho, wo = _im2col(x, stride)
    w2 = jnp.transpose(w, (2, 3, 1, 0)).reshape(9 * i, o)
    a2 = add.reshape(-1, o) if add is not None else None
    y = _matmul_fused(cols.reshape(-1, 9 * i), w2, *bn_args, add=a2)
    return y.reshape(nb, ho, wo, o)


def _seq_x(x, ps):
    for p in ps:
        x = _cbl_x(x, p)
    return x


def _up2(x):
    return jnp.repeat(jnp.repeat(x, 2, axis=1), 2, axis=2)


def _detect_x(x, dp):
    y = _cbl_x(x, dp["c"])
    w = dp["d"]["w"][:, :, 0, 0].T
    i, o = w.shape
    pad_n = 128 - o
    wp = jnp.pad(w, ((0, 0), (0, pad_n)))
    zeros = jnp.zeros((128,), _F32)
    ones = jnp.ones((128,), _F32)
    bp = jnp.pad(dp["d"]["b"], (0, pad_n))
    nb, h, wd, _ = y.shape
    z = _matmul_fused(y.reshape(-1, i), wp, zeros, ones, ones, bp, leaky=False)
    return z.reshape(nb, h, wd, 128)[..., :o]


def kernel(x, params):
    xh = jnp.transpose(x, (0, 2, 3, 1))
    t = _cbl_x(xh, params["stem"])
    feats = []
    for st in params["stages"]:
        t = _cbl_x(t, st["down"], stride=2)
        for blk in st["blocks"]:
            t = _cbl_x(_cbl_x(t, blk["c1"]), blk["c2"], add=t)
        feats.append(t)
    f13, f26, f52 = feats[-1], feats[-2], feats[-3]

    x1 = _seq_x(f13, params["s1"])
    p13 = _detect_x(x1, params["s1d"])
    x1u = _up2(_cbl_x(x1, params["up1"]))
    x2 = _seq_x(jnp.concatenate([x1u, f26], axis=-1), params["s2"])
    p26 = _detect_x(x2, params["s2d"])
    x2u = _up2(_cbl_x(x2, params["up2"]))
    x3 = _seq_x(jnp.concatenate([x2u, f52], axis=-1), params["s3"])
    p52 = _detect_x(x3, params["s3d"])
    return p13, p26, p52
